# baseline (speedup 1.0000x reference)
import functools

import jax
import jax.numpy as jnp
from jax import lax
from jax.experimental import pallas as pl
from jax.experimental.pallas import tpu as pltpu
from jax.experimental.pallas import tpu_sc as plsc

B, N, C = 4, 4096, 14
NSAMPLE = 512
KNN_POINTS = [16, 32]
MLP_LIST = [[64, 64, 128], [128, 128, 256]]
EMBED_DIM = 256
EPS = 1e-5
KMAX = 32
GI = 2

def _fps_body(ptex_ref, xyz_ref, out_ref):
    lin = (jax.lax.broadcasted_iota(jnp.int32, (8, 512), 0) * 512
           + jax.lax.broadcasted_iota(jnp.int32, (8, 512), 1))
    Xs = [xyz_ref[b, 0] for b in range(B)]
    Ys = [xyz_ref[b, 1] for b in range(B)]
    Zs = [xyz_ref[b, 2] for b in range(B)]

    def step(s, carry):
        dists, fars = carry
        new_dists = []
        new_fars = []
        for b in range(B):
            far = fars[b]
            row = ptex_ref[b, pl.ds(far, 1), :]
            out_ref[b, pl.ds(s, 1), :] = row
            cx = row[0, 0]
            cy = row[0, 1]
            cz = row[0, 2]
            dx = Xs[b] - cx
            dy = Ys[b] - cy
            dz = Zs[b] - cz
            d2 = (dx * dx + dy * dy) + dz * dz
            dist = jnp.minimum(dists[b], d2)
            m = jnp.max(dist)
            cand = jnp.where(dist == m, lin, jnp.int32(1 << 30))
            far2 = jnp.min(cand)
            new_dists.append(dist)
            new_fars.append(far2)
        return tuple(new_dists), tuple(new_fars)

    init = (tuple(jnp.full((8, 512), 1e10, dtype=jnp.float32) for _ in range(B)),
            tuple(jnp.int32(0) for _ in range(B)))
    jax.lax.fori_loop(0, NSAMPLE, step, init)


def _fps_pallas(points, aff):
    ptex = jnp.concatenate(
        [points, aff, jnp.zeros((B, N, 1), jnp.float32)], axis=-1)
    xyz = jnp.transpose(points[:, :, :3], (0, 2, 1)).reshape(B, 3, 8, 512)
    out = pl.pallas_call(
        _fps_body,
        out_shape=jax.ShapeDtypeStruct((B, NSAMPLE, 16), jnp.float32),
    )(ptex, xyz)
    return out[:, :, :14], out[:, :, 14:15]


def _mask_body(affn_ref, affs_ref, mn_ref, ms_ref):
    HALF = jnp.int32(0x3F000000)
    for b in range(B):
        an = affn_ref[b]
        asn = affs_ref[b]
        for (a, k, out_ref) in ((an, 409, mn_ref), (asn, 51, ms_ref)):
            cnt_gt = jnp.sum((a > HALF).astype(jnp.float32))
            adjust = cnt_gt < 0.5
            kf = jnp.float32(k)

            def bstep(i, carry):
                lo, hi = carry
                mid = (lo + hi + 1) // 2
                c = jnp.sum((a >= mid).astype(jnp.float32))
                pred = c >= kf
                lo2 = jnp.where(pred, mid, lo)
                hi2 = jnp.where(pred, hi, mid - 1)
                return lo2, hi2

            lo, hi = jax.lax.fori_loop(
                0, 31, bstep, (jnp.int32(0), jnp.int32(0x3F800000)))
            thr = jnp.where(adjust, lo, HALF)
            out_ref[b] = (a >= thr).astype(jnp.float32)


def _mask_pallas(points_aff_map, sampled_aff):
    affn = jax.lax.bitcast_convert_type(points_aff_map.reshape(B, 8, 512), jnp.int32)
    affs = jax.lax.bitcast_convert_type(sampled_aff.reshape(B, 8, 64), jnp.int32)
    mn, ms = pl.pallas_call(
        _mask_body,
        out_shape=[jax.ShapeDtypeStruct((B, 8, 512), jnp.float32),
                   jax.ShapeDtypeStruct((B, 8, 64), jnp.float32)],
    )(affn, affs)
    return mn.reshape(B, N, 1), ms.reshape(B, NSAMPLE, 1)


def _knn_body(q_ref, p_ref, idx_ref):
    lin = jax.lax.broadcasted_iota(jnp.int32, (8, 4096), 1)
    kiota = jax.lax.broadcasted_iota(jnp.int32, (8, KMAX), 1)

    def bf16r(x):
        return x.astype(jnp.bfloat16).astype(jnp.float32)

    for b in range(B):
        px = p_ref[b, 0]
        py = p_ref[b, 1]
        pz = p_ref[b, 2]
        pn = (px * px + py * py) + pz * pz
        pxh = bf16r(px)
        pyh = bf16r(py)
        pzh = bf16r(pz)

        def keys_for(t):
            qx = q_ref[b, 0, pl.ds(t * 8, 8), :]
            qy = q_ref[b, 1, pl.ds(t * 8, 8), :]
            qz = q_ref[b, 2, pl.ds(t * 8, 8), :]
            qn = (qx * qx + qy * qy) + qz * qz
            cross = (bf16r(qx) * pxh + bf16r(qy) * pyh) + bf16r(qz) * pzh
            return jnp.maximum((qn + pn) - 2.0 * cross, 0.0)

        def tile(t, _):
            ds = [keys_for(t * GI + g) for g in range(GI)]
            accs = [jnp.zeros((8, KMAX), jnp.int32) for _ in range(GI)]
            for r in range(KMAX):
                for g in range(GI):
                    m = jnp.min(ds[g], axis=1, keepdims=True)
                    cand = jnp.where(ds[g] == m, lin, 1 << 30)
                    ix = jnp.min(cand, axis=1, keepdims=True)
                    ds[g] = jnp.where(lin == ix, 3.0e38, ds[g])
                    accs[g] = jnp.where(kiota == r, ix, accs[g])
            for g in range(GI):
                idx_ref[b, pl.ds((t * GI + g) * 8, 8), :] = accs[g] + b * N
            return 0

        jax.lax.fori_loop(0, 64 // GI, tile, 0)


def _knn_topk_pallas(query, points):
    q = jnp.transpose(query, (0, 2, 1))[..., None]
    p = jnp.broadcast_to(jnp.transpose(points, (0, 2, 1))[:, :, None, :],
                         (B, 3, 8, N))
    idx = pl.pallas_call(
        _knn_body,
        out_shape=jax.ShapeDtypeStruct((B, NSAMPLE, KMAX), jnp.int32),
    )(q, p)
    return idx


def _gather_rows(tables, idx_flat):
    R = idx_flat.shape[0]
    info = plsc.get_sparse_core_info()
    NW = info.num_cores * info.num_subcores
    b_per_w = R // NW
    CHUNK = 128
    nch = b_per_w // CHUNK
    idx2 = idx_flat.reshape(NW * nch, CHUNK)
    mesh = plsc.VectorSubcoreMesh(core_axis_name="c", subcore_axis_name="s")

    @functools.partial(
        pl.kernel, mesh=mesh,
        out_type=jax.ShapeDtypeStruct((R, 128), jnp.float32),
        scratch_types=[
            pltpu.VMEM((nch, CHUNK), jnp.int32),
            pltpu.VMEM((2, CHUNK, 128), jnp.float32),
            pltpu.SemaphoreType.DMA,
            pltpu.SemaphoreType.DMA,
            pltpu.SemaphoreType.DMA,
        ],
    )
    def gather_k(table_hbm, idx_hbm, out_hbm, idx_v, rows_v, gsem, osem0, osem1):
        wid = lax.axis_index("s") * info.num_cores + lax.axis_index("c")
        pltpu.sync_copy(idx_hbm.at[pl.ds(wid * nch, nch), :], idx_v)
        osems = [osem0, osem1]
        out_cps = []
        for j in range(nch):
            buf = rows_v.at[j % 2]
            if j >= 2:
                out_cps[j - 2].wait()
            g = pltpu.make_async_copy(table_hbm.at[idx_v.at[j]], buf, gsem)
            g.start()
            g.wait()
            oc = pltpu.make_async_copy(
                buf, out_hbm.at[pl.ds((wid * nch + j) * CHUNK, CHUNK), :],
                osems[j % 2])
            oc.start()
            out_cps.append(oc)
        out_cps[nch - 2].wait()
        out_cps[nch - 1].wait()

    return gather_k(tables, idx2)


def _make_mlp_body(R, k, cos, masked):
    TR = 4096
    nt = R // TR
    TQ = TR // k

    def body(g_ref, m_ref, w0_ref, w1f_ref, w1g_ref, w2f_ref, w2g_ref,
             cb0_ref, cb1_ref, cb2_ref,
             bw0_ref, bb0_ref, bw1_ref, bb1_ref, bw2_ref, bb2_ref,
             pf_ref, A, g_scr, dma_sem):
        wf_refs = [w0_ref, w1f_ref, w2f_ref]
        wg_refs = [None, w1g_ref, w2g_ref]
        cb_refs = [cb0_ref, cb1_ref, cb2_ref]
        bw_refs = [bw0_ref, bw1_ref, bw2_ref]
        bb_refs = [bb0_ref, bb1_ref, bb2_ref]

        def mask_tile(t):
            mq = m_ref[pl.ds(t * TQ, TQ), :]
            return jnp.broadcast_to(
                mq.reshape(TQ, 1, 1), (TQ, k, 1)).reshape(TR, 1)

        def g_tile(t):
            cp = pltpu.make_async_copy(
                g_ref.at[pl.ds(t * TR, TR), :], g_scr, dma_sem)
            cp.start()
            cp.wait()
            return g_scr[...]

        def conv_tile(t, layer):
            g_t = g_tile(t)
            if masked:
                m_t = mask_tile(t)
                g_t = g_t * m_t
            if layer == 0:
                z = jnp.dot(g_t, wf_refs[0][...],
                            preferred_element_type=jnp.float32)
            else:
                a_t = A[pl.ds(t * TR, TR), :]
                if masked:
                    a_t = a_t * m_t
                z = (jnp.dot(a_t, wf_refs[layer][...],
                             preferred_element_type=jnp.float32)
                     + jnp.dot(g_t, wg_refs[layer][...],
                               preferred_element_type=jnp.float32))
            return z + cb_refs[layer][...]

        for layer in range(3):
            Co = cos[layer]

            def stat_step(t, carry):
                s1, s2, s0 = carry
                z = conv_tile(t, layer)
                if masked:
                    m_t = mask_tile(t)
                    s1 = s1 + jnp.sum(z * m_t, axis=0, keepdims=True)
                    s2 = s2 + jnp.sum(z * z * m_t, axis=0, keepdims=True)
                    s0 = s0 + jnp.sum(m_t)
                else:
                    s1 = s1 + jnp.sum(z, axis=0, keepdims=True)
                    s2 = s2 + jnp.sum(z * z, axis=0, keepdims=True)
                return s1, s2, s0

            init = (jnp.zeros((1, Co), jnp.float32),
                    jnp.zeros((1, Co), jnp.float32), jnp.float32(0.0))
            s1, s2, s0 = jax.lax.fori_loop(0, nt, stat_step, init)

            if masked:
                vc = s0 / k + EPS
                mean = s1 / vc
                var = (s2 - 2.0 * mean * s1 + mean * mean * s0) / vc
            else:
                mean = s1 / R
                var = (s2 - 2.0 * mean * s1 + mean * mean * R) / R
            sq = jnp.sqrt(var + EPS)

            def apply_step(t, _):
                z = conv_tile(t, layer)
                xn = (z - mean) / sq
                if masked:
                    xn = xn * bw_refs[layer][...] + bb_refs[layer][...]
                f = jnp.maximum(xn, 0.0)
                if layer < 2:
                    if Co < 128:
                        f2 = jnp.concatenate(
                            [f, jnp.zeros((TR, 128 - Co), jnp.float32)], axis=1)
                    else:
                        f2 = f
                    A[pl.ds(t * TR, TR), :] = f2
                else:
                    fp = jnp.max(f.reshape(TR // k, k, Co), axis=1)
                    pf_ref[pl.ds(t * (TR // k), TR // k), :] = fp
                return 0

            jax.lax.fori_loop(0, nt, apply_step, 0)

    return body


def _mlp_pallas(g_rows, m_rows, params, block, masked):
    cos = MLP_LIST[block]
    k = KNN_POINTS[block]
    R = B * NSAMPLE * k

    def wpad(w, rows):
        wt = w.T
        return jnp.concatenate(
            [wt, jnp.zeros((rows - wt.shape[0], wt.shape[1]), jnp.float32)],
            axis=0)

    w0 = params["conv_w_%d_0" % block]
    w1 = params["conv_w_%d_1" % block]
    w2 = params["conv_w_%d_2" % block]
    co0, co1, co2 = cos
    w0p = wpad(w0, 16)
    w1f = wpad(w1[:, :co0], 128)
    w1g = wpad(w1[:, co0:], 16)
    w2f = wpad(w2[:, :co1], 128)
    w2g = wpad(w2[:, co1:], 16)
    args = [g_rows, m_rows, w0p, w1f, w1g, w2f, w2g]
    for j in range(3):
        args.append(params["conv_b_%d_%d" % (block, j)].reshape(1, cos[j]))
    for j in range(3):
        args.append(params["bn_w_%d_%d" % (block, j)].reshape(1, cos[j]))
        args.append(params["bn_b_%d_%d" % (block, j)].reshape(1, cos[j]))
    nin = len(args)
    in_specs = [pl.BlockSpec(memory_space=pl.ANY)] + [
        pl.BlockSpec(memory_space=pltpu.VMEM) for _ in range(nin - 1)]
    pf = pl.pallas_call(
        _make_mlp_body(R, k, cos, masked),
        out_shape=jax.ShapeDtypeStruct((B * NSAMPLE, cos[2]), jnp.float32),
        in_specs=in_specs,
        scratch_shapes=[pltpu.VMEM((R, 128), jnp.float32),
                        pltpu.VMEM((4096, 16), jnp.float32),
                        pltpu.SemaphoreType.DMA],
    )(*args)
    return pf


def _embed_body(pf_ref, af_ref, w_ref, b_ref, out_ref):
    pe = jnp.dot(pf_ref[...], w_ref[...], preferred_element_type=jnp.float32)
    ae = jnp.dot(af_ref[...], w_ref[...], preferred_element_type=jnp.float32)
    out_ref[...] = jnp.concatenate([pe, ae], axis=-1) + jnp.concatenate(
        [b_ref[...], b_ref[...]], axis=-1)


def _embed_pallas(pf, af, ew, eb):
    total = pf.shape[1]
    out = pl.pallas_call(
        _embed_body,
        out_shape=jax.ShapeDtypeStruct((B * NSAMPLE, 2 * EMBED_DIM), jnp.float32),
    )(pf, af, ew.T, eb.reshape(1, EMBED_DIM))
    return out.reshape(B, NSAMPLE, 2 * EMBED_DIM)


@jax.jit
def kernel(points, points_aff_map, mask,
           conv_w_0_0, conv_b_0_0, bn_w_0_0, bn_b_0_0,
           conv_w_0_1, conv_b_0_1, bn_w_0_1, bn_b_0_1,
           conv_w_0_2, conv_b_0_2, bn_w_0_2, bn_b_0_2,
           conv_w_1_0, conv_b_1_0, bn_w_1_0, bn_b_1_0,
           conv_w_1_1, conv_b_1_1, bn_w_1_1, bn_b_1_1,
           conv_w_1_2, conv_b_1_2, bn_w_1_2, bn_b_1_2,
           embed_w, embed_b):
    params = {
        "conv_w_0_0": conv_w_0_0, "conv_b_0_0": conv_b_0_0,
        "bn_w_0_0": bn_w_0_0, "bn_b_0_0": bn_b_0_0,
        "conv_w_0_1": conv_w_0_1, "conv_b_0_1": conv_b_0_1,
        "bn_w_0_1": bn_w_0_1, "bn_b_0_1": bn_b_0_1,
        "conv_w_0_2": conv_w_0_2, "conv_b_0_2": conv_b_0_2,
        "bn_w_0_2": bn_w_0_2, "bn_b_0_2": bn_b_0_2,
        "conv_w_1_0": conv_w_1_0, "conv_b_1_0": conv_b_1_0,
        "bn_w_1_0": bn_w_1_0, "bn_b_1_0": bn_b_1_0,
        "conv_w_1_1": conv_w_1_1, "conv_b_1_1": conv_b_1_1,
        "bn_w_1_1": bn_w_1_1, "bn_b_1_1": bn_b_1_1,
        "conv_w_1_2": conv_w_1_2, "conv_b_1_2": conv_b_1_2,
        "bn_w_1_2": bn_w_1_2, "bn_b_1_2": bn_b_1_2,
    }
    sampled_points, sampled_aff_mask = _fps_pallas(points, points_aff_map)
    binary_points, binary_sampled = _mask_pallas(points_aff_map, sampled_aff_mask)
    sampled_aff = sampled_points * binary_sampled
    points_aff = points * binary_points

    idx_p = _knn_topk_pallas(sampled_points[:, :, :3], points[:, :, :3])
    coords_a = jnp.where(binary_points != 0, points_aff[:, :, :3],
                         jnp.float32(1e9))
    idx_a = _knn_topk_pallas(sampled_aff[:, :, :3], coords_a)

    zpad = jnp.zeros((B, N, 114), jnp.float32)
    tables = jnp.concatenate([
        jnp.concatenate([points, zpad], axis=-1).reshape(B * N, 128),
        jnp.concatenate([points_aff, zpad], axis=-1).reshape(B * N, 128),
    ], axis=0)
    idx_all = jnp.concatenate([idx_p.reshape(-1), idx_a.reshape(-1) + B * N])
    rows = _gather_rows(tables, idx_all)
    RK = B * NSAMPLE * KMAX
    gp32 = rows[:RK, :16]
    ga32 = rows[RK:, :16]

    m_q = binary_sampled.reshape(B * NSAMPLE, 1)

    g16_p = gp32.reshape(B * NSAMPLE, KMAX, 16)[:, :16, :].reshape(-1, 16)
    g32_p = gp32
    g16_a = ga32.reshape(B * NSAMPLE, KMAX, 16)[:, :16, :].reshape(-1, 16)
    g32_a = ga32

    pf0 = _mlp_pallas(g16_p, m_q, params, 0, masked=False)
    pf1 = _mlp_pallas(g32_p, m_q, params, 1, masked=False)
    af0 = _mlp_pallas(g16_a, m_q, params, 0, masked=True)
    af1 = _mlp_pallas(g32_a, m_q, params, 1, masked=True)

    pf = jnp.concatenate([pf0, pf1], axis=1)
    af = jnp.concatenate([af0, af1], axis=1)
    return _embed_pallas(pf, af, embed_w, embed_b)


# speedup vs baseline: 1.3655x; 1.3655x over previous
import functools

import jax
import jax.numpy as jnp
from jax import lax
from jax.experimental import pallas as pl
from jax.experimental.pallas import tpu as pltpu
from jax.experimental.pallas import tpu_sc as plsc

B, N, C = 4, 4096, 14
NSAMPLE = 512
KNN_POINTS = [16, 32]
MLP_LIST = [[64, 64, 128], [128, 128, 256]]
EMBED_DIM = 256
EPS = 1e-5
KMAX = 32
GI = 4

def _fps_body(ptex_ref, xyz_ref, out_ref):
    lin = (jax.lax.broadcasted_iota(jnp.int32, (8, 512), 0) * 512
           + jax.lax.broadcasted_iota(jnp.int32, (8, 512), 1))
    Xs = [xyz_ref[b, 0] for b in range(B)]
    Ys = [xyz_ref[b, 1] for b in range(B)]
    Zs = [xyz_ref[b, 2] for b in range(B)]

    def step(s, carry):
        dists, fars = carry
        new_dists = []
        new_fars = []
        for b in range(B):
            far = fars[b]
            row = ptex_ref[b, pl.ds(far, 1), :]
            out_ref[b, pl.ds(s, 1), :] = row
            cx = row[0, 0]
            cy = row[0, 1]
            cz = row[0, 2]
            dx = Xs[b] - cx
            dy = Ys[b] - cy
            dz = Zs[b] - cz
            d2 = (dx * dx + dy * dy) + dz * dz
            dist = jnp.minimum(dists[b], d2)
            m = jnp.max(dist)
            cand = jnp.where(dist == m, lin, jnp.int32(1 << 30))
            far2 = jnp.min(cand)
            new_dists.append(dist)
            new_fars.append(far2)
        return tuple(new_dists), tuple(new_fars)

    init = (tuple(jnp.full((8, 512), 1e10, dtype=jnp.float32) for _ in range(B)),
            tuple(jnp.int32(0) for _ in range(B)))
    jax.lax.fori_loop(0, NSAMPLE, step, init)


def _fps_pallas(points, aff):
    ptex = jnp.concatenate(
        [points, aff, jnp.zeros((B, N, 1), jnp.float32)], axis=-1)
    xyz = jnp.transpose(points[:, :, :3], (0, 2, 1)).reshape(B, 3, 8, 512)
    out = pl.pallas_call(
        _fps_body,
        out_shape=jax.ShapeDtypeStruct((B, NSAMPLE, 16), jnp.float32),
    )(ptex, xyz)
    return out[:, :, :14], out[:, :, 14:15]


def _mask_body(affn_ref, affs_ref, mn_ref, ms_ref):
    HALF = jnp.int32(0x3F000000)
    for b in range(B):
        an = affn_ref[b]
        asn = affs_ref[b]
        for (a, k, out_ref) in ((an, 409, mn_ref), (asn, 51, ms_ref)):
            cnt_gt = jnp.sum((a > HALF).astype(jnp.float32))
            adjust = cnt_gt < 0.5
            kf = jnp.float32(k)

            def bstep(i, carry):
                lo, hi = carry
                mid = (lo + hi + 1) // 2
                c = jnp.sum((a >= mid).astype(jnp.float32))
                pred = c >= kf
                lo2 = jnp.where(pred, mid, lo)
                hi2 = jnp.where(pred, hi, mid - 1)
                return lo2, hi2

            lo, hi = jax.lax.fori_loop(
                0, 31, bstep, (jnp.int32(0), jnp.int32(0x3F800000)))
            thr = jnp.where(adjust, lo, HALF)
            out_ref[b] = (a >= thr).astype(jnp.float32)


def _mask_pallas(points_aff_map, sampled_aff):
    affn = jax.lax.bitcast_convert_type(points_aff_map.reshape(B, 8, 512), jnp.int32)
    affs = jax.lax.bitcast_convert_type(sampled_aff.reshape(B, 8, 64), jnp.int32)
    mn, ms = pl.pallas_call(
        _mask_body,
        out_shape=[jax.ShapeDtypeStruct((B, 8, 512), jnp.float32),
                   jax.ShapeDtypeStruct((B, 8, 64), jnp.float32)],
    )(affn, affs)
    return mn.reshape(B, N, 1), ms.reshape(B, NSAMPLE, 1)


def _knn_body(q_ref, p_ref, idx_ref):
    lin = jax.lax.broadcasted_iota(jnp.int32, (8, 4096), 1)
    kiota = jax.lax.broadcasted_iota(jnp.int32, (8, KMAX), 1)

    def bf16r(x):
        return x.astype(jnp.bfloat16).astype(jnp.float32)

    for b in range(B):
        px = p_ref[b, 0]
        py = p_ref[b, 1]
        pz = p_ref[b, 2]
        pn = (px * px + py * py) + pz * pz
        pxh = bf16r(px)
        pyh = bf16r(py)
        pzh = bf16r(pz)

        def keys_for(t):
            qx = q_ref[b, 0, pl.ds(t * 8, 8), :]
            qy = q_ref[b, 1, pl.ds(t * 8, 8), :]
            qz = q_ref[b, 2, pl.ds(t * 8, 8), :]
            qn = (qx * qx + qy * qy) + qz * qz
            cross = (bf16r(qx) * pxh + bf16r(qy) * pyh) + bf16r(qz) * pzh
            return jnp.maximum((qn + pn) - 2.0 * cross, 0.0)

        def tile(t, _):
            ds = [keys_for(t * GI + g) for g in range(GI)]
            accs = [jnp.zeros((8, KMAX), jnp.int32) for _ in range(GI)]
            for r in range(KMAX):
                for g in range(GI):
                    m = jnp.min(ds[g], axis=1, keepdims=True)
                    cand = jnp.where(ds[g] == m, lin, 1 << 30)
                    ix = jnp.min(cand, axis=1, keepdims=True)
                    ds[g] = jnp.where(lin == ix, 3.0e38, ds[g])
                    accs[g] = jnp.where(kiota == r, ix, accs[g])
            for g in range(GI):
                idx_ref[b, pl.ds((t * GI + g) * 8, 8), :] = accs[g] + b * N
            return 0

        jax.lax.fori_loop(0, 64 // GI, tile, 0)


def _knn_topk_pallas(query, points):
    q = jnp.transpose(query, (0, 2, 1))[..., None]
    p = jnp.broadcast_to(jnp.transpose(points, (0, 2, 1))[:, :, None, :],
                         (B, 3, 8, N))
    idx = pl.pallas_call(
        _knn_body,
        out_shape=jax.ShapeDtypeStruct((B, NSAMPLE, KMAX), jnp.int32),
    )(q, p)
    return idx


def _gather_rows(tables, idx_flat):
    R = idx_flat.shape[0]
    info = plsc.get_sparse_core_info()
    NW = info.num_cores * info.num_subcores
    b_per_w = R // NW
    CHUNK = 128
    nch = b_per_w // CHUNK
    idx2 = idx_flat.reshape(NW * nch, CHUNK)
    mesh = plsc.VectorSubcoreMesh(core_axis_name="c", subcore_axis_name="s")

    @functools.partial(
        pl.kernel, mesh=mesh,
        out_type=jax.ShapeDtypeStruct((R, 128), jnp.float32),
        scratch_types=[
            pltpu.VMEM((nch, CHUNK), jnp.int32),
            pltpu.VMEM((2, CHUNK, 128), jnp.float32),
            pltpu.SemaphoreType.DMA,
            pltpu.SemaphoreType.DMA,
            pltpu.SemaphoreType.DMA,
        ],
    )
    def gather_k(table_hbm, idx_hbm, out_hbm, idx_v, rows_v, gsem, osem0, osem1):
        wid = lax.axis_index("s") * info.num_cores + lax.axis_index("c")
        pltpu.sync_copy(idx_hbm.at[pl.ds(wid * nch, nch), :], idx_v)
        osems = [osem0, osem1]
        out_cps = []
        for j in range(nch):
            buf = rows_v.at[j % 2]
            if j >= 2:
                out_cps[j - 2].wait()
            g = pltpu.make_async_copy(table_hbm.at[idx_v.at[j]], buf, gsem)
            g.start()
            g.wait()
            oc = pltpu.make_async_copy(
                buf, out_hbm.at[pl.ds((wid * nch + j) * CHUNK, CHUNK), :],
                osems[j % 2])
            oc.start()
            out_cps.append(oc)
        out_cps[nch - 2].wait()
        out_cps[nch - 1].wait()

    return gather_k(tables, idx2)


def _make_mlp_body(R, k, cos, masked):
    TR = 4096
    nt = R // TR
    TQ = TR // k

    def body(g_ref, m_ref, w0_ref, w1f_ref, w1g_ref, w2f_ref, w2g_ref,
             cb0_ref, cb1_ref, cb2_ref,
             bw0_ref, bb0_ref, bw1_ref, bb1_ref, bw2_ref, bb2_ref,
             pf_ref, A, g_scr, dma_sem):
        wf_refs = [w0_ref, w1f_ref, w2f_ref]
        wg_refs = [None, w1g_ref, w2g_ref]
        cb_refs = [cb0_ref, cb1_ref, cb2_ref]
        bw_refs = [bw0_ref, bw1_ref, bw2_ref]
        bb_refs = [bb0_ref, bb1_ref, bb2_ref]

        def mask_tile(t):
            mq = m_ref[pl.ds(t * TQ, TQ), :]
            return jnp.broadcast_to(
                mq.reshape(TQ, 1, 1), (TQ, k, 1)).reshape(TR, 1)

        def g_tile(t):
            cp = pltpu.make_async_copy(
                g_ref.at[pl.ds(t * TR, TR), :], g_scr, dma_sem)
            cp.start()
            cp.wait()
            return g_scr[...]

        def conv_tile(t, layer):
            g_t = g_tile(t)
            if masked:
                m_t = mask_tile(t)
                g_t = g_t * m_t
            if layer == 0:
                z = jnp.dot(g_t, wf_refs[0][...],
                            preferred_element_type=jnp.float32)
            else:
                a_t = A[pl.ds(t * TR, TR), :]
                if masked:
                    a_t = a_t * m_t
                z = (jnp.dot(a_t, wf_refs[layer][...],
                             preferred_element_type=jnp.float32)
                     + jnp.dot(g_t, wg_refs[layer][...],
                               preferred_element_type=jnp.float32))
            return z + cb_refs[layer][...]

        for layer in range(3):
            Co = cos[layer]

            def stat_step(t, carry):
                s1, s2, s0 = carry
                z = conv_tile(t, layer)
                if masked:
                    m_t = mask_tile(t)
                    s1 = s1 + jnp.sum(z * m_t, axis=0, keepdims=True)
                    s2 = s2 + jnp.sum(z * z * m_t, axis=0, keepdims=True)
                    s0 = s0 + jnp.sum(m_t)
                else:
                    s1 = s1 + jnp.sum(z, axis=0, keepdims=True)
                    s2 = s2 + jnp.sum(z * z, axis=0, keepdims=True)
                return s1, s2, s0

            init = (jnp.zeros((1, Co), jnp.float32),
                    jnp.zeros((1, Co), jnp.float32), jnp.float32(0.0))
            s1, s2, s0 = jax.lax.fori_loop(0, nt, stat_step, init)

            if masked:
                vc = s0 / k + EPS
                mean = s1 / vc
                var = (s2 - 2.0 * mean * s1 + mean * mean * s0) / vc
            else:
                mean = s1 / R
                var = (s2 - 2.0 * mean * s1 + mean * mean * R) / R
            sq = jnp.sqrt(var + EPS)

            def apply_step(t, _):
                z = conv_tile(t, layer)
                xn = (z - mean) / sq
                if masked:
                    xn = xn * bw_refs[layer][...] + bb_refs[layer][...]
                f = jnp.maximum(xn, 0.0)
                if layer < 2:
                    if Co < 128:
                        f2 = jnp.concatenate(
                            [f, jnp.zeros((TR, 128 - Co), jnp.float32)], axis=1)
                    else:
                        f2 = f
                    A[pl.ds(t * TR, TR), :] = f2
                else:
                    fp = jnp.max(f.reshape(TR // k, k, Co), axis=1)
                    pf_ref[pl.ds(t * (TR // k), TR // k), :] = fp
                return 0

            jax.lax.fori_loop(0, nt, apply_step, 0)

    return body


def _mlp_pallas(g_rows, m_rows, params, block, masked):
    cos = MLP_LIST[block]
    k = KNN_POINTS[block]
    R = B * NSAMPLE * k

    def wpad(w, rows):
        wt = w.T
        return jnp.concatenate(
            [wt, jnp.zeros((rows - wt.shape[0], wt.shape[1]), jnp.float32)],
            axis=0)

    w0 = params["conv_w_%d_0" % block]
    w1 = params["conv_w_%d_1" % block]
    w2 = params["conv_w_%d_2" % block]
    co0, co1, co2 = cos
    w0p = wpad(w0, 16)
    w1f = wpad(w1[:, :co0], 128)
    w1g = wpad(w1[:, co0:], 16)
    w2f = wpad(w2[:, :co1], 128)
    w2g = wpad(w2[:, co1:], 16)
    args = [g_rows, m_rows, w0p, w1f, w1g, w2f, w2g]
    for j in range(3):
        args.append(params["conv_b_%d_%d" % (block, j)].reshape(1, cos[j]))
    for j in range(3):
        args.append(params["bn_w_%d_%d" % (block, j)].reshape(1, cos[j]))
        args.append(params["bn_b_%d_%d" % (block, j)].reshape(1, cos[j]))
    nin = len(args)
    in_specs = [pl.BlockSpec(memory_space=pl.ANY)] + [
        pl.BlockSpec(memory_space=pltpu.VMEM) for _ in range(nin - 1)]
    pf = pl.pallas_call(
        _make_mlp_body(R, k, cos, masked),
        out_shape=jax.ShapeDtypeStruct((B * NSAMPLE, cos[2]), jnp.float32),
        in_specs=in_specs,
        scratch_shapes=[pltpu.VMEM((R, 128), jnp.float32),
                        pltpu.VMEM((4096, 16), jnp.float32),
                        pltpu.SemaphoreType.DMA],
    )(*args)
    return pf


def _embed_body(pf_ref, af_ref, w_ref, b_ref, out_ref):
    pe = jnp.dot(pf_ref[...], w_ref[...], preferred_element_type=jnp.float32)
    ae = jnp.dot(af_ref[...], w_ref[...], preferred_element_type=jnp.float32)
    out_ref[...] = jnp.concatenate([pe, ae], axis=-1) + jnp.concatenate(
        [b_ref[...], b_ref[...]], axis=-1)


def _embed_pallas(pf, af, ew, eb):
    total = pf.shape[1]
    out = pl.pallas_call(
        _embed_body,
        out_shape=jax.ShapeDtypeStruct((B * NSAMPLE, 2 * EMBED_DIM), jnp.float32),
    )(pf, af, ew.T, eb.reshape(1, EMBED_DIM))
    return out.reshape(B, NSAMPLE, 2 * EMBED_DIM)


@jax.jit
def kernel(points, points_aff_map, mask,
           conv_w_0_0, conv_b_0_0, bn_w_0_0, bn_b_0_0,
           conv_w_0_1, conv_b_0_1, bn_w_0_1, bn_b_0_1,
           conv_w_0_2, conv_b_0_2, bn_w_0_2, bn_b_0_2,
           conv_w_1_0, conv_b_1_0, bn_w_1_0, bn_b_1_0,
           conv_w_1_1, conv_b_1_1, bn_w_1_1, bn_b_1_1,
           conv_w_1_2, conv_b_1_2, bn_w_1_2, bn_b_1_2,
           embed_w, embed_b):
    params = {
        "conv_w_0_0": conv_w_0_0, "conv_b_0_0": conv_b_0_0,
        "bn_w_0_0": bn_w_0_0, "bn_b_0_0": bn_b_0_0,
        "conv_w_0_1": conv_w_0_1, "conv_b_0_1": conv_b_0_1,
        "bn_w_0_1": bn_w_0_1, "bn_b_0_1": bn_b_0_1,
        "conv_w_0_2": conv_w_0_2, "conv_b_0_2": conv_b_0_2,
        "bn_w_0_2": bn_w_0_2, "bn_b_0_2": bn_b_0_2,
        "conv_w_1_0": conv_w_1_0, "conv_b_1_0": conv_b_1_0,
        "bn_w_1_0": bn_w_1_0, "bn_b_1_0": bn_b_1_0,
        "conv_w_1_1": conv_w_1_1, "conv_b_1_1": conv_b_1_1,
        "bn_w_1_1": bn_w_1_1, "bn_b_1_1": bn_b_1_1,
        "conv_w_1_2": conv_w_1_2, "conv_b_1_2": conv_b_1_2,
        "bn_w_1_2": bn_w_1_2, "bn_b_1_2": bn_b_1_2,
    }
    sampled_points, sampled_aff_mask = _fps_pallas(points, points_aff_map)
    binary_points, binary_sampled = _mask_pallas(points_aff_map, sampled_aff_mask)
    sampled_aff = sampled_points * binary_sampled
    points_aff = points * binary_points

    idx_p = _knn_topk_pallas(sampled_points[:, :, :3], points[:, :, :3])
    coords_a = jnp.where(binary_points != 0, points_aff[:, :, :3],
                         jnp.float32(1e9))
    idx_a = _knn_topk_pallas(sampled_aff[:, :, :3], coords_a)

    zpad = jnp.zeros((B, N, 114), jnp.float32)
    tables = jnp.concatenate([
        jnp.concatenate([points, zpad], axis=-1).reshape(B * N, 128),
        jnp.concatenate([points_aff, zpad], axis=-1).reshape(B * N, 128),
    ], axis=0)
    idx_all = jnp.concatenate([idx_p.reshape(-1), idx_a.reshape(-1) + B * N])
    rows = _gather_rows(tables, idx_all)
    RK = B * NSAMPLE * KMAX
    gp32 = rows[:RK, :16]
    ga32 = rows[RK:, :16]

    m_q = binary_sampled.reshape(B * NSAMPLE, 1)

    g16_p = gp32.reshape(B * NSAMPLE, KMAX, 16)[:, :16, :].reshape(-1, 16)
    g32_p = gp32
    g16_a = ga32.reshape(B * NSAMPLE, KMAX, 16)[:, :16, :].reshape(-1, 16)
    g32_a = ga32

    pf0 = _mlp_pallas(g16_p, m_q, params, 0, masked=False)
    pf1 = _mlp_pallas(g32_p, m_q, params, 1, masked=False)
    af0 = _mlp_pallas(g16_a, m_q, params, 0, masked=True)
    af1 = _mlp_pallas(g32_a, m_q, params, 1, masked=True)

    pf = jnp.concatenate([pf0, pf1], axis=1)
    af = jnp.concatenate([af0, af1], axis=1)
    return _embed_pallas(pf, af, embed_w, embed_b)


# speedup vs baseline: 1.5953x; 1.1683x over previous
import functools

import jax
import jax.numpy as jnp
from jax import lax
from jax.experimental import pallas as pl
from jax.experimental.pallas import tpu as pltpu
from jax.experimental.pallas import tpu_sc as plsc

B, N, C = 4, 4096, 14
NSAMPLE = 512
KNN_POINTS = [16, 32]
MLP_LIST = [[64, 64, 128], [128, 128, 256]]
EMBED_DIM = 256
EPS = 1e-5
KMAX = 32
GI = 8

def _fps_body(ptex_ref, xyz_ref, out_ref):
    lin = (jax.lax.broadcasted_iota(jnp.int32, (8, 512), 0) * 512
           + jax.lax.broadcasted_iota(jnp.int32, (8, 512), 1))
    Xs = [xyz_ref[b, 0] for b in range(B)]
    Ys = [xyz_ref[b, 1] for b in range(B)]
    Zs = [xyz_ref[b, 2] for b in range(B)]

    def step(s, carry):
        dists, fars = carry
        new_dists = []
        new_fars = []
        for b in range(B):
            far = fars[b]
            row = ptex_ref[b, pl.ds(far, 1), :]
            out_ref[b, pl.ds(s, 1), :] = row
            cx = row[0, 0]
            cy = row[0, 1]
            cz = row[0, 2]
            dx = Xs[b] - cx
            dy = Ys[b] - cy
            dz = Zs[b] - cz
            d2 = (dx * dx + dy * dy) + dz * dz
            dist = jnp.minimum(dists[b], d2)
            m = jnp.max(dist)
            cand = jnp.where(dist == m, lin, jnp.int32(1 << 30))
            far2 = jnp.min(cand)
            new_dists.append(dist)
            new_fars.append(far2)
        return tuple(new_dists), tuple(new_fars)

    init = (tuple(jnp.full((8, 512), 1e10, dtype=jnp.float32) for _ in range(B)),
            tuple(jnp.int32(0) for _ in range(B)))
    jax.lax.fori_loop(0, NSAMPLE, step, init)


def _fps_pallas(points, aff):
    ptex = jnp.concatenate(
        [points, aff, jnp.zeros((B, N, 1), jnp.float32)], axis=-1)
    xyz = jnp.transpose(points[:, :, :3], (0, 2, 1)).reshape(B, 3, 8, 512)
    out = pl.pallas_call(
        _fps_body,
        out_shape=jax.ShapeDtypeStruct((B, NSAMPLE, 16), jnp.float32),
    )(ptex, xyz)
    return out[:, :, :14], out[:, :, 14:15]


def _mask_body(affn_ref, affs_ref, mn_ref, ms_ref):
    HALF = jnp.int32(0x3F000000)
    for b in range(B):
        an = affn_ref[b]
        asn = affs_ref[b]
        for (a, k, out_ref) in ((an, 409, mn_ref), (asn, 51, ms_ref)):
            cnt_gt = jnp.sum((a > HALF).astype(jnp.float32))
            adjust = cnt_gt < 0.5
            kf = jnp.float32(k)

            def bstep(i, carry):
                lo, hi = carry
                mid = (lo + hi + 1) // 2
                c = jnp.sum((a >= mid).astype(jnp.float32))
                pred = c >= kf
                lo2 = jnp.where(pred, mid, lo)
                hi2 = jnp.where(pred, hi, mid - 1)
                return lo2, hi2

            lo, hi = jax.lax.fori_loop(
                0, 31, bstep, (jnp.int32(0), jnp.int32(0x3F800000)))
            thr = jnp.where(adjust, lo, HALF)
            out_ref[b] = (a >= thr).astype(jnp.float32)


def _mask_pallas(points_aff_map, sampled_aff):
    affn = jax.lax.bitcast_convert_type(points_aff_map.reshape(B, 8, 512), jnp.int32)
    affs = jax.lax.bitcast_convert_type(sampled_aff.reshape(B, 8, 64), jnp.int32)
    mn, ms = pl.pallas_call(
        _mask_body,
        out_shape=[jax.ShapeDtypeStruct((B, 8, 512), jnp.float32),
                   jax.ShapeDtypeStruct((B, 8, 64), jnp.float32)],
    )(affn, affs)
    return mn.reshape(B, N, 1), ms.reshape(B, NSAMPLE, 1)


def _knn_body(q_ref, p_ref, idx_ref):
    lin = jax.lax.broadcasted_iota(jnp.int32, (8, 4096), 1)
    kiota = jax.lax.broadcasted_iota(jnp.int32, (8, KMAX), 1)

    def bf16r(x):
        return x.astype(jnp.bfloat16).astype(jnp.float32)

    for b in range(B):
        px = p_ref[b, 0]
        py = p_ref[b, 1]
        pz = p_ref[b, 2]
        pn = (px * px + py * py) + pz * pz
        pxh = bf16r(px)
        pyh = bf16r(py)
        pzh = bf16r(pz)

        def keys_for(t):
            qx = q_ref[b, 0, pl.ds(t * 8, 8), :]
            qy = q_ref[b, 1, pl.ds(t * 8, 8), :]
            qz = q_ref[b, 2, pl.ds(t * 8, 8), :]
            qn = (qx * qx + qy * qy) + qz * qz
            cross = (bf16r(qx) * pxh + bf16r(qy) * pyh) + bf16r(qz) * pzh
            return jnp.maximum((qn + pn) - 2.0 * cross, 0.0)

        def tile(t, _):
            ds = [keys_for(t * GI + g) for g in range(GI)]
            accs = [jnp.zeros((8, KMAX), jnp.int32) for _ in range(GI)]
            for r in range(KMAX):
                for g in range(GI):
                    m = jnp.min(ds[g], axis=1, keepdims=True)
                    cand = jnp.where(ds[g] == m, lin, 1 << 30)
                    ix = jnp.min(cand, axis=1, keepdims=True)
                    ds[g] = jnp.where(lin == ix, 3.0e38, ds[g])
                    accs[g] = jnp.where(kiota == r, ix, accs[g])
            for g in range(GI):
                idx_ref[b, pl.ds((t * GI + g) * 8, 8), :] = accs[g] + b * N
            return 0

        jax.lax.fori_loop(0, 64 // GI, tile, 0)


def _knn_topk_pallas(query, points):
    q = jnp.transpose(query, (0, 2, 1))[..., None]
    p = jnp.broadcast_to(jnp.transpose(points, (0, 2, 1))[:, :, None, :],
                         (B, 3, 8, N))
    idx = pl.pallas_call(
        _knn_body,
        out_shape=jax.ShapeDtypeStruct((B, NSAMPLE, KMAX), jnp.int32),
    )(q, p)
    return idx


def _gather_rows(tables, idx_flat):
    R = idx_flat.shape[0]
    info = plsc.get_sparse_core_info()
    NW = info.num_cores * info.num_subcores
    b_per_w = R // NW
    CHUNK = 128
    nch = b_per_w // CHUNK
    idx2 = idx_flat.reshape(NW * nch, CHUNK)
    mesh = plsc.VectorSubcoreMesh(core_axis_name="c", subcore_axis_name="s")

    @functools.partial(
        pl.kernel, mesh=mesh,
        out_type=jax.ShapeDtypeStruct((R, 128), jnp.float32),
        scratch_types=[
            pltpu.VMEM((nch, CHUNK), jnp.int32),
            pltpu.VMEM((2, CHUNK, 128), jnp.float32),
            pltpu.SemaphoreType.DMA,
            pltpu.SemaphoreType.DMA,
            pltpu.SemaphoreType.DMA,
        ],
    )
    def gather_k(table_hbm, idx_hbm, out_hbm, idx_v, rows_v, gsem, osem0, osem1):
        wid = lax.axis_index("s") * info.num_cores + lax.axis_index("c")
        pltpu.sync_copy(idx_hbm.at[pl.ds(wid * nch, nch), :], idx_v)
        osems = [osem0, osem1]
        out_cps = []
        for j in range(nch):
            buf = rows_v.at[j % 2]
            if j >= 2:
                out_cps[j - 2].wait()
            g = pltpu.make_async_copy(table_hbm.at[idx_v.at[j]], buf, gsem)
            g.start()
            g.wait()
            oc = pltpu.make_async_copy(
                buf, out_hbm.at[pl.ds((wid * nch + j) * CHUNK, CHUNK), :],
                osems[j % 2])
            oc.start()
            out_cps.append(oc)
        out_cps[nch - 2].wait()
        out_cps[nch - 1].wait()

    return gather_k(tables, idx2)


def _make_mlp_body(R, k, cos, masked):
    TR = 4096
    nt = R // TR
    TQ = TR // k

    def body(g_ref, m_ref, w0_ref, w1f_ref, w1g_ref, w2f_ref, w2g_ref,
             cb0_ref, cb1_ref, cb2_ref,
             bw0_ref, bb0_ref, bw1_ref, bb1_ref, bw2_ref, bb2_ref,
             pf_ref, A, g_scr, dma_sem):
        wf_refs = [w0_ref, w1f_ref, w2f_ref]
        wg_refs = [None, w1g_ref, w2g_ref]
        cb_refs = [cb0_ref, cb1_ref, cb2_ref]
        bw_refs = [bw0_ref, bw1_ref, bw2_ref]
        bb_refs = [bb0_ref, bb1_ref, bb2_ref]

        def mask_tile(t):
            mq = m_ref[pl.ds(t * TQ, TQ), :]
            return jnp.broadcast_to(
                mq.reshape(TQ, 1, 1), (TQ, k, 1)).reshape(TR, 1)

        def g_tile(t):
            cp = pltpu.make_async_copy(
                g_ref.at[pl.ds(t * TR, TR), :], g_scr, dma_sem)
            cp.start()
            cp.wait()
            return g_scr[...]

        def conv_tile(t, layer):
            g_t = g_tile(t)
            if masked:
                m_t = mask_tile(t)
                g_t = g_t * m_t
            if layer == 0:
                z = jnp.dot(g_t, wf_refs[0][...],
                            preferred_element_type=jnp.float32)
            else:
                a_t = A[pl.ds(t * TR, TR), :]
                if masked:
                    a_t = a_t * m_t
                z = (jnp.dot(a_t, wf_refs[layer][...],
                             preferred_element_type=jnp.float32)
                     + jnp.dot(g_t, wg_refs[layer][...],
                               preferred_element_type=jnp.float32))
            return z + cb_refs[layer][...]

        for layer in range(3):
            Co = cos[layer]

            def stat_step(t, carry):
                s1, s2, s0 = carry
                z = conv_tile(t, layer)
                if masked:
                    m_t = mask_tile(t)
                    s1 = s1 + jnp.sum(z * m_t, axis=0, keepdims=True)
                    s2 = s2 + jnp.sum(z * z * m_t, axis=0, keepdims=True)
                    s0 = s0 + jnp.sum(m_t)
                else:
                    s1 = s1 + jnp.sum(z, axis=0, keepdims=True)
                    s2 = s2 + jnp.sum(z * z, axis=0, keepdims=True)
                return s1, s2, s0

            init = (jnp.zeros((1, Co), jnp.float32),
                    jnp.zeros((1, Co), jnp.float32), jnp.float32(0.0))
            s1, s2, s0 = jax.lax.fori_loop(0, nt, stat_step, init)

            if masked:
                vc = s0 / k + EPS
                mean = s1 / vc
                var = (s2 - 2.0 * mean * s1 + mean * mean * s0) / vc
            else:
                mean = s1 / R
                var = (s2 - 2.0 * mean * s1 + mean * mean * R) / R
            sq = jnp.sqrt(var + EPS)

            def apply_step(t, _):
                z = conv_tile(t, layer)
                xn = (z - mean) / sq
                if masked:
                    xn = xn * bw_refs[layer][...] + bb_refs[layer][...]
                f = jnp.maximum(xn, 0.0)
                if layer < 2:
                    if Co < 128:
                        f2 = jnp.concatenate(
                            [f, jnp.zeros((TR, 128 - Co), jnp.float32)], axis=1)
                    else:
                        f2 = f
                    A[pl.ds(t * TR, TR), :] = f2
                else:
                    fp = jnp.max(f.reshape(TR // k, k, Co), axis=1)
                    pf_ref[pl.ds(t * (TR // k), TR // k), :] = fp
                return 0

            jax.lax.fori_loop(0, nt, apply_step, 0)

    return body


def _mlp_pallas(g_rows, m_rows, params, block, masked):
    cos = MLP_LIST[block]
    k = KNN_POINTS[block]
    R = B * NSAMPLE * k

    def wpad(w, rows):
        wt = w.T
        return jnp.concatenate(
            [wt, jnp.zeros((rows - wt.shape[0], wt.shape[1]), jnp.float32)],
            axis=0)

    w0 = params["conv_w_%d_0" % block]
    w1 = params["conv_w_%d_1" % block]
    w2 = params["conv_w_%d_2" % block]
    co0, co1, co2 = cos
    w0p = wpad(w0, 16)
    w1f = wpad(w1[:, :co0], 128)
    w1g = wpad(w1[:, co0:], 16)
    w2f = wpad(w2[:, :co1], 128)
    w2g = wpad(w2[:, co1:], 16)
    args = [g_rows, m_rows, w0p, w1f, w1g, w2f, w2g]
    for j in range(3):
        args.append(params["conv_b_%d_%d" % (block, j)].reshape(1, cos[j]))
    for j in range(3):
        args.append(params["bn_w_%d_%d" % (block, j)].reshape(1, cos[j]))
        args.append(params["bn_b_%d_%d" % (block, j)].reshape(1, cos[j]))
    nin = len(args)
    in_specs = [pl.BlockSpec(memory_space=pl.ANY)] + [
        pl.BlockSpec(memory_space=pltpu.VMEM) for _ in range(nin - 1)]
    pf = pl.pallas_call(
        _make_mlp_body(R, k, cos, masked),
        out_shape=jax.ShapeDtypeStruct((B * NSAMPLE, cos[2]), jnp.float32),
        in_specs=in_specs,
        scratch_shapes=[pltpu.VMEM((R, 128), jnp.float32),
                        pltpu.VMEM((4096, 16), jnp.float32),
                        pltpu.SemaphoreType.DMA],
    )(*args)
    return pf


def _embed_body(pf_ref, af_ref, w_ref, b_ref, out_ref):
    pe = jnp.dot(pf_ref[...], w_ref[...], preferred_element_type=jnp.float32)
    ae = jnp.dot(af_ref[...], w_ref[...], preferred_element_type=jnp.float32)
    out_ref[...] = jnp.concatenate([pe, ae], axis=-1) + jnp.concatenate(
        [b_ref[...], b_ref[...]], axis=-1)


def _embed_pallas(pf, af, ew, eb):
    total = pf.shape[1]
    out = pl.pallas_call(
        _embed_body,
        out_shape=jax.ShapeDtypeStruct((B * NSAMPLE, 2 * EMBED_DIM), jnp.float32),
    )(pf, af, ew.T, eb.reshape(1, EMBED_DIM))
    return out.reshape(B, NSAMPLE, 2 * EMBED_DIM)


@jax.jit
def kernel(points, points_aff_map, mask,
           conv_w_0_0, conv_b_0_0, bn_w_0_0, bn_b_0_0,
           conv_w_0_1, conv_b_0_1, bn_w_0_1, bn_b_0_1,
           conv_w_0_2, conv_b_0_2, bn_w_0_2, bn_b_0_2,
           conv_w_1_0, conv_b_1_0, bn_w_1_0, bn_b_1_0,
           conv_w_1_1, conv_b_1_1, bn_w_1_1, bn_b_1_1,
           conv_w_1_2, conv_b_1_2, bn_w_1_2, bn_b_1_2,
           embed_w, embed_b):
    params = {
        "conv_w_0_0": conv_w_0_0, "conv_b_0_0": conv_b_0_0,
        "bn_w_0_0": bn_w_0_0, "bn_b_0_0": bn_b_0_0,
        "conv_w_0_1": conv_w_0_1, "conv_b_0_1": conv_b_0_1,
        "bn_w_0_1": bn_w_0_1, "bn_b_0_1": bn_b_0_1,
        "conv_w_0_2": conv_w_0_2, "conv_b_0_2": conv_b_0_2,
        "bn_w_0_2": bn_w_0_2, "bn_b_0_2": bn_b_0_2,
        "conv_w_1_0": conv_w_1_0, "conv_b_1_0": conv_b_1_0,
        "bn_w_1_0": bn_w_1_0, "bn_b_1_0": bn_b_1_0,
        "conv_w_1_1": conv_w_1_1, "conv_b_1_1": conv_b_1_1,
        "bn_w_1_1": bn_w_1_1, "bn_b_1_1": bn_b_1_1,
        "conv_w_1_2": conv_w_1_2, "conv_b_1_2": conv_b_1_2,
        "bn_w_1_2": bn_w_1_2, "bn_b_1_2": bn_b_1_2,
    }
    sampled_points, sampled_aff_mask = _fps_pallas(points, points_aff_map)
    binary_points, binary_sampled = _mask_pallas(points_aff_map, sampled_aff_mask)
    sampled_aff = sampled_points * binary_sampled
    points_aff = points * binary_points

    idx_p = _knn_topk_pallas(sampled_points[:, :, :3], points[:, :, :3])
    coords_a = jnp.where(binary_points != 0, points_aff[:, :, :3],
                         jnp.float32(1e9))
    idx_a = _knn_topk_pallas(sampled_aff[:, :, :3], coords_a)

    zpad = jnp.zeros((B, N, 114), jnp.float32)
    tables = jnp.concatenate([
        jnp.concatenate([points, zpad], axis=-1).reshape(B * N, 128),
        jnp.concatenate([points_aff, zpad], axis=-1).reshape(B * N, 128),
    ], axis=0)
    idx_all = jnp.concatenate([idx_p.reshape(-1), idx_a.reshape(-1) + B * N])
    rows = _gather_rows(tables, idx_all)
    RK = B * NSAMPLE * KMAX
    gp32 = rows[:RK, :16]
    ga32 = rows[RK:, :16]

    m_q = binary_sampled.reshape(B * NSAMPLE, 1)

    g16_p = gp32.reshape(B * NSAMPLE, KMAX, 16)[:, :16, :].reshape(-1, 16)
    g32_p = gp32
    g16_a = ga32.reshape(B * NSAMPLE, KMAX, 16)[:, :16, :].reshape(-1, 16)
    g32_a = ga32

    pf0 = _mlp_pallas(g16_p, m_q, params, 0, masked=False)
    pf1 = _mlp_pallas(g32_p, m_q, params, 1, masked=False)
    af0 = _mlp_pallas(g16_a, m_q, params, 0, masked=True)
    af1 = _mlp_pallas(g32_a, m_q, params, 1, masked=True)

    pf = jnp.concatenate([pf0, pf1], axis=1)
    af = jnp.concatenate([af0, af1], axis=1)
    return _embed_pallas(pf, af, embed_w, embed_b)
